# baseline (speedup 1.0000x reference)
import functools

import jax
import jax.numpy as jnp
from jax import lax
from jax.experimental import pallas as pl
from jax.experimental.pallas import tpu as pltpu
from jax.experimental.pallas import tpu_sc as plsc

_B = 8
_M = 2048
_D = 1024
_TOTAL = 8192
_NW = 32
_VR = _TOTAL // _NW
_C = 32
_NCV = _VR // _C
_NBUF = 3
_ZC = 256


def _make_sc_kernel():
  mesh = plsc.VectorSubcoreMesh(core_axis_name="c", subcore_axis_name="s")

  @functools.partial(
      pl.kernel,
      mesh=mesh,
      out_type=jax.ShapeDtypeStruct((_B * _M, _D), jnp.float32),
      scratch_types=[
          pltpu.VMEM((16,), jnp.int32),
          pltpu.VMEM((_NCV, _C), jnp.int32),
          pltpu.VMEM((1, 16), jnp.int32),
          pltpu.VMEM((_NBUF, _C, _D), jnp.float32),
          pltpu.VMEM((16, _D), jnp.float32),
          pltpu.SemaphoreType.DMA,
          pltpu.SemaphoreType.DMA,
          pltpu.SemaphoreType.DMA,
      ],
  )
  def k(flat_hbm, cu_hbm, out_hbm,
        cu_v, idxv, idxb, bufs, zbuf, gsem, ssem, bsem):
    wid = lax.axis_index("s") * 2 + lax.axis_index("c")
    pltpu.sync_copy(cu_hbm, cu_v.at[pl.ds(0, _B + 1)])

    iota = lax.iota(jnp.int32, 16)
    cu_vec = cu_v[pl.ds(0, 16)]
    cus = [cu_vec[i] for i in range(_B + 1)]
    zero = jnp.int32(0)

    vbase = wid * _VR

    def valid_idx(g, carry):
      iv = vbase + g * 16 + iota
      bm = jnp.where(iv >= cus[1], jnp.int32(_M), zero)
      cu_b = jnp.where(iv >= cus[1], cus[1] - cus[0], zero)
      for t in range(2, _B):
        bm = bm + jnp.where(iv >= cus[t], jnp.int32(_M), zero)
        cu_b = cu_b + jnp.where(iv >= cus[t], cus[t] - cus[t - 1], zero)
      grp = _C // 16
      idxv[g // grp, pl.ds((g % grp) * 16, 16)] = bm + iv - cu_b
      return carry

    lax.fori_loop(0, _VR // 16, valid_idx, zero)

    def gather(c, slot):
      pltpu.async_copy(
          flat_hbm.at[pl.ds(vbase + c * _C, _C)], bufs.at[slot], gsem)

    def wait_gather(c, slot):
      pltpu.make_async_copy(
          flat_hbm.at[pl.ds(vbase + c * _C, _C)], bufs.at[slot], gsem).wait()

    def scatter(c, slot):
      pltpu.async_copy(bufs.at[slot], out_hbm.at[idxv.at[c]], ssem)

    def wait_scatter(c, slot):
      pltpu.make_async_copy(
          bufs.at[slot], out_hbm.at[idxv.at[c]], ssem).wait()

    for c in range(_NBUF):
      gather(c, c)

    b4 = wid // 4
    lenb = jnp.where(b4 == 0, cus[1] - cus[0], zero)
    for t in range(1, _B):
      lenb = lenb + jnp.where(b4 == t, cus[t + 1] - cus[t], zero)
    nb = (8 - jnp.mod(lenb, 8)) % 8
    do_boundary = jnp.logical_and(jnp.mod(wid, 4) == 0, nb > 0)

    @pl.when(do_boundary)
    def _():
      def zrow(r, carry):
        for w in range(_D // 16):
          zbuf[r, pl.ds(w * 16, 16)] = jnp.zeros((16,), jnp.float32)
        return carry

      lax.fori_loop(0, 16, zrow, zero)
      idxb[0, pl.ds(0, 16)] = b4 * _M + lenb + jnp.minimum(iota, nb - 1)
      pltpu.async_copy(zbuf, out_hbm.at[idxb.at[0]], bsem)

    def pipe(c, slot):
      wait_gather(c, slot)
      scatter(c, slot)

      @pl.when(c + _NBUF < _NCV)
      def _():
        wait_scatter(c, slot)
        gather(c + _NBUF, slot)

      return jnp.where(slot == _NBUF - 1, 0, slot + 1)

    lax.fori_loop(0, _NCV, pipe, zero)

    def drain(c, carry):
      wait_scatter(c, jnp.mod(c, _NBUF))
      return carry

    lax.fori_loop(_NCV - _NBUF, _NCV, drain, zero)

    @pl.when(do_boundary)
    def _():
      pltpu.make_async_copy(zbuf, out_hbm.at[idxb.at[0]], bsem).wait()

  return k


def _tc_zero_body(cu_ref, init_ref, out_ref, zc, sem):
  del init_ref
  zc[...] = jnp.zeros((_ZC, _D), jnp.float32)
  for fire in (True, False):
    for b in range(_B):
      lenb = cu_ref[b + 1] - cu_ref[b]
      start = ((lenb + 7) // 8) * 8
      length = _M - start
      big = length // _ZC
      end = (b + 1) * _M

      def zchunk(i, carry):
        cp = pltpu.make_async_copy(
            zc,
            out_ref.at[pl.ds(pl.multiple_of(end - (i + 1) * _ZC, 8), _ZC)],
            sem)
        if fire:
          cp.start()
        else:
          cp.wait()
        return carry

      lax.fori_loop(0, big, zchunk, jnp.int32(0))

      pos = end - big * _ZC
      rem = length - big * _ZC
      kk = _ZC // 2
      while kk >= 8:
        k = kk
        hit = (rem & k) != 0
        pos = jnp.where(hit, pos - k, pos)

        @pl.when(hit)
        def _(pos=pos, k=k):
          cp = pltpu.make_async_copy(
              zc.at[pl.ds(0, k)],
              out_ref.at[pl.ds(pl.multiple_of(pos, 8), k)], sem)
          if fire:
            cp.start()
          else:
            cp.wait()

        kk //= 2


_sc_valid = _make_sc_kernel()

_tc_zero = pl.pallas_call(
    _tc_zero_body,
    out_shape=jax.ShapeDtypeStruct((_B * _M, _D), jnp.float32),
    in_specs=[
        pl.BlockSpec(memory_space=pltpu.SMEM),
        pl.BlockSpec(memory_space=pl.ANY),
    ],
    out_specs=pl.BlockSpec(memory_space=pl.ANY),
    input_output_aliases={1: 0},
    scratch_shapes=[
        pltpu.VMEM((_ZC, _D), jnp.float32),
        pltpu.SemaphoreType.DMA,
    ],
)


def _tc_mask_body(cu_ref, mask_ref):
  m = lax.broadcasted_iota(jnp.int32, (_B, _M), 1)
  row = lax.broadcasted_iota(jnp.int32, (_B, _M), 0)
  acc = jnp.zeros((_B, _M), jnp.bool_)
  for b in range(_B):
    lenb = cu_ref[b + 1] - cu_ref[b]
    acc = jnp.logical_or(acc, jnp.logical_and(row == b, m >= lenb))
  mask_ref[...] = acc


_tc_mask = pl.pallas_call(
    _tc_mask_body,
    out_shape=jax.ShapeDtypeStruct((_B, _M), jnp.bool_),
    in_specs=[pl.BlockSpec(memory_space=pltpu.SMEM)],
)


@jax.jit
def kernel(flat, cu_seqlens):
  cu = cu_seqlens.astype(jnp.int32)
  scattered = _sc_valid(flat, cu)
  mask = _tc_mask(cu)
  out = _tc_zero(cu, scattered)
  return out.reshape(_B, _M, _D), mask


# speedup vs baseline: 1.0657x; 1.0657x over previous
import functools

import jax
import jax.numpy as jnp
from jax import lax
from jax.experimental import pallas as pl
from jax.experimental.pallas import tpu as pltpu
from jax.experimental.pallas import tpu_sc as plsc

_B = 8
_M = 2048
_D = 1024
_TOTAL = 8192
_NW = 32
_VR = _TOTAL // _NW
_C = 32
_NCV = _VR // _C
_NBUF = 3
_ZC = 256


def _make_sc_kernel():
  mesh = plsc.VectorSubcoreMesh(core_axis_name="c", subcore_axis_name="s")

  @functools.partial(
      pl.kernel,
      mesh=mesh,
      out_type=(),
      scratch_types=[
          pltpu.VMEM((16,), jnp.int32),
          pltpu.VMEM((_NCV, _C), jnp.int32),
          pltpu.VMEM((_NBUF, _C, _D), jnp.float32),
          pltpu.SemaphoreType.DMA,
          pltpu.SemaphoreType.DMA,
      ],
  )
  def k(flat_hbm, cu_hbm, out_hbm, cu_v, idxv, bufs, gsem, ssem):
    wid = lax.axis_index("s") * 2 + lax.axis_index("c")
    pltpu.sync_copy(cu_hbm, cu_v.at[pl.ds(0, _B + 1)])

    iota = lax.iota(jnp.int32, 16)
    cu_vec = cu_v[pl.ds(0, 16)]
    cus = [cu_vec[i] for i in range(_B + 1)]
    zero = jnp.int32(0)

    vbase = wid * _VR

    def valid_idx(g, carry):
      iv = vbase + g * 16 + iota
      bm = jnp.where(iv >= cus[1], jnp.int32(_M), zero)
      cu_b = jnp.where(iv >= cus[1], cus[1] - cus[0], zero)
      for t in range(2, _B):
        bm = bm + jnp.where(iv >= cus[t], jnp.int32(_M), zero)
        cu_b = cu_b + jnp.where(iv >= cus[t], cus[t] - cus[t - 1], zero)
      grp = _C // 16
      idxv[g // grp, pl.ds((g % grp) * 16, 16)] = bm + iv - cu_b
      return carry

    lax.fori_loop(0, _VR // 16, valid_idx, zero)

    def gather(c, slot):
      pltpu.async_copy(
          flat_hbm.at[pl.ds(vbase + c * _C, _C)], bufs.at[slot], gsem)

    def wait_gather(c, slot):
      pltpu.make_async_copy(
          flat_hbm.at[pl.ds(vbase + c * _C, _C)], bufs.at[slot], gsem).wait()

    def scatter(c, slot):
      pltpu.async_copy(bufs.at[slot], out_hbm.at[idxv.at[c]], ssem)

    def wait_scatter(c, slot):
      pltpu.make_async_copy(
          bufs.at[slot], out_hbm.at[idxv.at[c]], ssem).wait()

    for c in range(_NBUF):
      gather(c, c)

    def pipe(c, slot):
      wait_gather(c, slot)
      scatter(c, slot)

      @pl.when(c + _NBUF < _NCV)
      def _():
        wait_scatter(c, slot)
        gather(c + _NBUF, slot)

      return jnp.where(slot == _NBUF - 1, 0, slot + 1)

    lax.fori_loop(0, _NCV, pipe, zero)

    def drain(c, carry):
      wait_scatter(c, jnp.mod(c, _NBUF))
      return carry

    lax.fori_loop(_NCV - _NBUF, _NCV, drain, zero)

  return k


def _tc_zero_body(cu_ref, out_ref, zc, sem):
  zc[...] = jnp.zeros((_ZC, _D), jnp.float32)
  for fire in (True, False):
    for b in range(_B):
      lenb = cu_ref[b + 1] - cu_ref[b]
      start = (lenb // 8) * 8
      length = _M - start
      big = length // _ZC
      end = (b + 1) * _M

      def zchunk(i, carry):
        cp = pltpu.make_async_copy(
            zc,
            out_ref.at[pl.ds(pl.multiple_of(end - (i + 1) * _ZC, 8), _ZC)],
            sem)
        if fire:
          cp.start()
        else:
          cp.wait()
        return carry

      lax.fori_loop(0, big, zchunk, jnp.int32(0))

      pos = end - big * _ZC
      rem = length - big * _ZC
      kk = _ZC // 2
      while kk >= 8:
        k = kk
        hit = (rem & k) != 0
        pos = jnp.where(hit, pos - k, pos)

        @pl.when(hit)
        def _(pos=pos, k=k):
          cp = pltpu.make_async_copy(
              zc.at[pl.ds(0, k)],
              out_ref.at[pl.ds(pl.multiple_of(pos, 8), k)], sem)
          if fire:
            cp.start()
          else:
            cp.wait()

        kk //= 2


_sc_valid = _make_sc_kernel()

_tc_zero = pl.pallas_call(
    _tc_zero_body,
    out_shape=jax.ShapeDtypeStruct((_B * _M, _D), jnp.float32),
    in_specs=[pl.BlockSpec(memory_space=pltpu.SMEM)],
    out_specs=pl.BlockSpec(memory_space=pl.ANY),
    scratch_shapes=[
        pltpu.VMEM((_ZC, _D), jnp.float32),
        pltpu.SemaphoreType.DMA,
    ],
)


def _tc_mask_body(cu_ref, mask_ref):
  m = lax.broadcasted_iota(jnp.int32, (_B, _M), 1)
  row = lax.broadcasted_iota(jnp.int32, (_B, _M), 0)
  acc = jnp.zeros((_B, _M), jnp.bool_)
  for b in range(_B):
    lenb = cu_ref[b + 1] - cu_ref[b]
    acc = jnp.logical_or(acc, jnp.logical_and(row == b, m >= lenb))
  mask_ref[...] = acc


_tc_mask = pl.pallas_call(
    _tc_mask_body,
    out_shape=jax.ShapeDtypeStruct((_B, _M), jnp.bool_),
    in_specs=[pl.BlockSpec(memory_space=pltpu.SMEM)],
)


@jax.jit
def kernel(flat, cu_seqlens):
  cu = cu_seqlens.astype(jnp.int32)
  init = _tc_zero(cu)
  ref = jax.new_ref(init)
  _sc_valid(flat, cu, ref)
  mask = _tc_mask(cu)
  return ref[...].reshape(_B, _M, _D), mask
